# baseline (speedup 1.0000x reference)
import functools

import jax
import jax.numpy as jnp
from jax import lax
from jax.experimental import pallas as pl
from jax.experimental.pallas import tpu as pltpu
from jax.experimental.pallas import tpu_sc as plsc

NC = 2
NS = 16
NW = NC * NS

_SLOPE = (1.0 / 8.0 + 1.0 / 3.0) / 2.0
_EPS = 1e-12


def _make_sc_spmm(V, W, E, N, B):
    EPW = E // NW
    assert EPW * NW == E and EPW % B == 0
    NCHUNK = EPW // B
    G = 50
    assert NCHUNK % G == 0 and G % 2 == 0
    NGROUP = NCHUNK // G
    ZR = 200
    NZB = N // ZR
    assert NZB * ZR == N
    KZ = (NZB + NS - 1) // NS

    mesh = plsc.VectorSubcoreMesh(core_axis_name="c", subcore_axis_name="s", num_cores=NC, num_subcores=NS)

    @functools.partial(
        pl.kernel,
        mesh=mesh,
        out_type=jax.ShapeDtypeStruct((NC, N, W), jnp.float32),
        scratch_types=[
            pltpu.VMEM((G, B), jnp.int32),
            pltpu.VMEM((G, B), jnp.int32),
            pltpu.VMEM((2, B, W), jnp.float32),
            pltpu.VMEM_SHARED((N, W), jnp.float32),
            pltpu.SemaphoreType.DMA,
            pltpu.SemaphoreType.DMA,
            pltpu.SemaphoreType.DMA,
            pltpu.SemaphoreType.DMA,
        ],
    )
    def spmm(table, gidx, sidx, ztab, out, gbuf, sbuf, rows, acc,
             gs0, gs1, ss0, ss1):
        cid = lax.axis_index("c")
        sid = lax.axis_index("s")
        wid = sid * NC + cid
        gs = (gs0, gs1)
        ss = (ss0, ss1)

        for k in range(KZ):
            blk = sid + k * NS

            @pl.when(blk < NZB)
            def _():
                pltpu.sync_copy(ztab.at[pl.ds(blk * ZR, ZR)],
                                acc.at[pl.ds(blk * ZR, ZR)])

        plsc.subcore_barrier()

        def w_gather(c, b):
            pltpu.make_async_copy(table.at[gbuf.at[c]], rows.at[b],
                                  gs[b]).wait()

        def i_gather(c, b):
            pltpu.async_copy(table.at[gbuf.at[c]], rows.at[b], gs[b])

        def i_scatter(c, b):
            pltpu.async_copy(rows.at[b], acc.at[sbuf.at[c]], ss[b], add=True)

        def w_scatter(c, b):
            pltpu.make_async_copy(rows.at[b], acc.at[sbuf.at[c]],
                                  ss[b]).wait()

        for g in range(NGROUP):
            pltpu.sync_copy(gidx.at[wid, g], gbuf)
            pltpu.sync_copy(sidx.at[wid, g], sbuf)
            i_gather(0, 0)

            def chunk_body(t, carry):
                for k in range(2):
                    c = t * 2 + k
                    w_gather(c, k)
                    i_scatter(c, k)

                    @pl.when(c >= 1)
                    def _():
                        w_scatter(c - 1, 1 - k)

                    @pl.when(c + 1 < G)
                    def _():
                        i_gather(c + 1, 1 - k)
                return carry
            lax.fori_loop(0, G // 2, chunk_body, 0)

            w_scatter(G - 1, 1)

        plsc.subcore_barrier()
        for k in range(KZ):
            blk = sid + k * NS

            @pl.when(blk < NZB)
            def _():
                pltpu.sync_copy(acc.at[pl.ds(blk * ZR, ZR)],
                                out.at[cid, pl.ds(blk * ZR, ZR)])

    return spmm


def _make_sc_pre(V, E, N, B):
    W = 128
    EPW = E // NW
    NCHUNK = EPW // B
    G = 50
    assert NCHUNK % G == 0 and G % 2 == 0
    NGROUP = NCHUNK // G
    ZR = 200
    NZB = N // ZR
    KZ = (NZB + NS - 1) // NS

    mesh = plsc.VectorSubcoreMesh(core_axis_name="c", subcore_axis_name="s", num_cores=NC, num_subcores=NS)

    @functools.partial(
        pl.kernel,
        mesh=mesh,
        out_type=(jax.ShapeDtypeStruct((NC, N, W), jnp.float32),
                  jax.ShapeDtypeStruct((NC, N, W), jnp.float32)),
        scratch_types=[
            pltpu.VMEM((G, B), jnp.int32),
            pltpu.VMEM((G, B), jnp.int32),
            pltpu.VMEM((2, B, W), jnp.float32),
            pltpu.VMEM_SHARED((N, W), jnp.float32),
            pltpu.SemaphoreType.DMA,
            pltpu.SemaphoreType.DMA,
            pltpu.SemaphoreType.DMA,
            pltpu.SemaphoreType.DMA,
        ],
    )
    def prek(table, gidx, sidx, ztab, out1, out2, gbuf, sbuf, rows, acc,
             gs0, gs1, ss0, ss1):
        cid = lax.axis_index("c")
        sid = lax.axis_index("s")
        wid = sid * NC + cid
        gs = (gs0, gs1)
        ss = (ss0, ss1)

        def zero_acc():
            for k in range(KZ):
                blk = sid + k * NS

                @pl.when(blk < NZB)
                def _():
                    pltpu.sync_copy(ztab.at[pl.ds(blk * ZR, ZR)],
                                    acc.at[pl.ds(blk * ZR, ZR)])

        def copy_out(out):
            for k in range(KZ):
                blk = sid + k * NS

                @pl.when(blk < NZB)
                def _():
                    pltpu.sync_copy(acc.at[pl.ds(blk * ZR, ZR)],
                                    out.at[cid, pl.ds(blk * ZR, ZR)])

        def w_gather(c, b):
            pltpu.make_async_copy(table.at[gbuf.at[c]], rows.at[b],
                                  gs[b]).wait()

        def i_gather(c, b):
            pltpu.async_copy(table.at[gbuf.at[c]], rows.at[b], gs[b])

        def i_scatter(c, b):
            pltpu.async_copy(rows.at[b], acc.at[sbuf.at[c]], ss[b], add=True)

        def w_scatter(c, b):
            pltpu.make_async_copy(rows.at[b], acc.at[sbuf.at[c]],
                                  ss[b]).wait()

        zero_acc()
        plsc.subcore_barrier()

        for g in range(NGROUP):
            pltpu.sync_copy(gidx.at[wid, g], gbuf)
            pltpu.sync_copy(sidx.at[wid, g], sbuf)
            i_gather(0, 0)

            def chunk_body(t, carry):
                for k in range(2):
                    c = t * 2 + k
                    w_gather(c, k)
                    i_scatter(c, k)

                    @pl.when(c >= 1)
                    def _():
                        w_scatter(c - 1, 1 - k)

                    @pl.when(c + 1 < G)
                    def _():
                        i_gather(c + 1, 1 - k)
                return carry
            lax.fori_loop(0, G // 2, chunk_body, 0)

            w_scatter(G - 1, 1)

        plsc.subcore_barrier()
        copy_out(out1)
        plsc.subcore_barrier()

        zero_acc()
        one16 = jnp.ones((16,), jnp.float32)
        for i in range(B):
            for j in range(W // 16):
                rows[0, i, pl.ds(16 * j, 16)] = one16
        plsc.subcore_barrier()

        def i_scatter1(c, b):
            pltpu.async_copy(rows.at[0], acc.at[sbuf.at[c]], ss[b], add=True)

        def w_scatter1(c, b):
            pltpu.make_async_copy(rows.at[0], acc.at[sbuf.at[c]],
                                  ss[b]).wait()

        for g in range(NGROUP):
            pltpu.sync_copy(sidx.at[wid, g], sbuf)

            def deg_body(t, carry):
                for k in range(2):
                    c = t * 2 + k

                    @pl.when(c >= 2)
                    def _():
                        w_scatter1(c - 2, k)

                    i_scatter1(c, k)
                return carry
            lax.fori_loop(0, G // 2, deg_body, 0)

            w_scatter1(G - 2, 0)
            w_scatter1(G - 1, 1)

        plsc.subcore_barrier()
        copy_out(out2)

    return prek


def _l2n(x):
    return x / (jnp.sqrt(jnp.sum(x * x, axis=1, keepdims=True)) + _EPS)


def _tc_pre_body(crel_ref, norm_ref, wn_ref, ent_ref, h0_ref, crnw_ref):
    h0_ref[...] = _l2n(ent_ref[...])
    c = (crel_ref[0] + crel_ref[1]) * norm_ref[...]
    crnw_ref[0] = jnp.dot(c, wn_ref[0], preferred_element_type=jnp.float32)
    crnw_ref[1] = jnp.dot(c, wn_ref[1], preferred_element_type=jnp.float32)


def _tc_precompute(crel_p, norm, w_neigh, emb_ent):
    N, H = emb_ent.shape
    RB = 2000
    grid = (N // RB,)
    return pl.pallas_call(
        _tc_pre_body,
        grid=grid,
        in_specs=[
            pl.BlockSpec((2, RB, H), lambda i: (0, i, 0)),
            pl.BlockSpec((RB, 1), lambda i: (i, 0)),
            pl.BlockSpec((2, H, H), lambda i: (0, 0, 0)),
            pl.BlockSpec((RB, H), lambda i: (i, 0)),
        ],
        out_specs=[
            pl.BlockSpec((RB, H), lambda i: (i, 0)),
            pl.BlockSpec((2, RB, H), lambda i: (0, i, 0)),
        ],
        out_shape=[
            jax.ShapeDtypeStruct((N, H), jnp.float32),
            jax.ShapeDtypeStruct((2, N, H), jnp.float32),
        ],
    )(crel_p, norm, w_neigh, emb_ent)


def _tc_layer_body(y_ref, norm_ref, crnw_ref, x_ref, wn_ref, ws_ref, o_ref):
    ysum = y_ref[0] + y_ref[1]
    z = jnp.dot(ysum, wn_ref[...], preferred_element_type=jnp.float32)
    z = z * norm_ref[...] - crnw_ref[...]
    z = z + jnp.dot(x_ref[...], ws_ref[...], preferred_element_type=jnp.float32)
    o_ref[...] = jnp.where(z >= 0, z, _SLOPE * z)


def _tc_layer(y_p, norm, crnw_l, x, wn_l, ws_l):
    N, H = x.shape
    RB = 2000
    grid = (N // RB,)
    return pl.pallas_call(
        _tc_layer_body,
        grid=grid,
        in_specs=[
            pl.BlockSpec((2, RB, H), lambda i: (0, i, 0)),
            pl.BlockSpec((RB, 1), lambda i: (i, 0)),
            pl.BlockSpec((RB, H), lambda i: (i, 0)),
            pl.BlockSpec((RB, H), lambda i: (i, 0)),
            pl.BlockSpec((H, H), lambda i: (0, 0)),
            pl.BlockSpec((H, H), lambda i: (0, 0)),
        ],
        out_specs=pl.BlockSpec((RB, H), lambda i: (i, 0)),
        out_shape=jax.ShapeDtypeStruct((N, H), jnp.float32),
    )(y_p, norm, crnw_l, x, wn_l, ws_l)


def _tc_layer_gru_body(y_ref, norm_ref, crnw_ref, x_ref, wn_ref, ws_ref,
                       h_ref, wih_ref, bih_ref, whh_ref, bhh_ref, o_ref):
    H = x_ref.shape[1]
    ysum = y_ref[0] + y_ref[1]
    z = jnp.dot(ysum, wn_ref[...], preferred_element_type=jnp.float32)
    z = z * norm_ref[...] - crnw_ref[...]
    z = z + jnp.dot(x_ref[...], ws_ref[...], preferred_element_type=jnp.float32)
    z = jnp.where(z >= 0, z, _SLOPE * z)
    xn = _l2n(z)
    h = h_ref[...]
    gi = lax.dot_general(xn, wih_ref[...], (((1,), (1,)), ((), ())),
                         preferred_element_type=jnp.float32) + bih_ref[...]
    gh = lax.dot_general(h, whh_ref[...], (((1,), (1,)), ((), ())),
                         preferred_element_type=jnp.float32) + bhh_ref[...]
    rg = jax.nn.sigmoid(gi[:, :H] + gh[:, :H])
    zg = jax.nn.sigmoid(gi[:, H:2 * H] + gh[:, H:2 * H])
    ng = jnp.tanh(gi[:, 2 * H:] + rg * gh[:, 2 * H:])
    hn = (1.0 - zg) * ng + zg * h
    o_ref[...] = _l2n(hn)


def _tc_layer_gru(y_p, norm, crnw_l, x, wn_l, ws_l, h, w_ih, b_ih, w_hh, b_hh):
    N, H = x.shape
    RB = 2000
    grid = (N // RB,)
    return pl.pallas_call(
        _tc_layer_gru_body,
        grid=grid,
        in_specs=[
            pl.BlockSpec((2, RB, H), lambda i: (0, i, 0)),
            pl.BlockSpec((RB, 1), lambda i: (i, 0)),
            pl.BlockSpec((RB, H), lambda i: (i, 0)),
            pl.BlockSpec((RB, H), lambda i: (i, 0)),
            pl.BlockSpec((H, H), lambda i: (0, 0)),
            pl.BlockSpec((H, H), lambda i: (0, 0)),
            pl.BlockSpec((RB, H), lambda i: (i, 0)),
            pl.BlockSpec((3 * H, H), lambda i: (0, 0)),
            pl.BlockSpec((1, 3 * H), lambda i: (0, 0)),
            pl.BlockSpec((3 * H, H), lambda i: (0, 0)),
            pl.BlockSpec((1, 3 * H), lambda i: (0, 0)),
        ],
        out_specs=pl.BlockSpec((RB, H), lambda i: (i, 0)),
        out_shape=jax.ShapeDtypeStruct((N, H), jnp.float32),
    )(y_p, norm, crnw_l, x, wn_l, ws_l, h, w_ih, b_ih, w_hh, b_hh)


def kernel(edge_index, edge_type, emb_ent, emb_rel, W_neigh, W_self,
           gru_W_ih, gru_b_ih, gru_W_hh, gru_b_hh):
    N, H = emb_ent.shape
    E = edge_type.shape[0]
    L = W_neigh.shape[0]
    R = emb_rel.shape[0] // 2
    T = 3

    src = edge_index[0]
    dst = edge_index[1]

    B_MAIN = 100
    G = 50
    NGRP = E // NW // B_MAIN // G
    src3 = src.reshape(NW, NGRP, G, B_MAIN)
    dst3 = dst.reshape(NW, NGRP, G, B_MAIN)

    ztab = jnp.zeros((N, H), jnp.float32)

    KREP = 16
    r_rep = jnp.tile(emb_rel[:R], (KREP, 1))
    et_rep = (jnp.arange(E, dtype=jnp.int32) % KREP) * R + edge_type
    et3 = et_rep.reshape(NW, NGRP, G, B_MAIN)

    pre_pass = _make_sc_pre(KREP * R, E, N, B_MAIN)
    spmm_main = _make_sc_spmm(N, H, E, N, B_MAIN)

    crel_p, deg_p = pre_pass(r_rep, et3, dst3, ztab)
    deg = deg_p[0, :, 0] + deg_p[1, :, 0]
    norm = (1.0 / jnp.maximum(deg, 1.0))[:, None]

    b_ih2 = gru_b_ih.reshape(1, 3 * H)
    b_hh2 = gru_b_hh.reshape(1, 3 * H)

    h, crnw = _tc_precompute(crel_p, norm, W_neigh, emb_ent)

    for _t in range(T):
        x = h
        for l in range(L - 1):
            y_p = spmm_main(x, src3, dst3, ztab)
            x = _tc_layer(y_p, norm, crnw[l], x, W_neigh[l], W_self[l])
        y_p = spmm_main(x, src3, dst3, ztab)
        h = _tc_layer_gru(y_p, norm, crnw[L - 1], x, W_neigh[L - 1],
                          W_self[L - 1], h, gru_W_ih, b_ih2, gru_W_hh, b_hh2)
    return h


# speedup vs baseline: 1.1826x; 1.1826x over previous
import functools

import jax
import jax.numpy as jnp
from jax import lax
from jax.experimental import pallas as pl
from jax.experimental.pallas import tpu as pltpu
from jax.experimental.pallas import tpu_sc as plsc

NC = 2
NS = 16
NW = NC * NS

_SLOPE = (1.0 / 8.0 + 1.0 / 3.0) / 2.0
_EPS = 1e-12


def _make_sc_spmm(V, W, E, N, B):
    EPW = E // NW
    assert EPW * NW == E and EPW % B == 0
    NCHUNK = EPW // B
    RING = 4
    G = 25
    assert NCHUNK % G == 0 and G % RING == 1
    NGROUP = NCHUNK // G
    ZR = 200
    NZB = N // ZR
    assert NZB * ZR == N
    KZ = (NZB + NS - 1) // NS

    mesh = plsc.VectorSubcoreMesh(core_axis_name="c", subcore_axis_name="s", num_cores=NC, num_subcores=NS)

    @functools.partial(
        pl.kernel,
        mesh=mesh,
        out_type=jax.ShapeDtypeStruct((NC, N, W), jnp.float32),
        scratch_types=[
            pltpu.VMEM((G, B), jnp.int32),
            pltpu.VMEM((G, B), jnp.int32),
            pltpu.VMEM((4, B, W), jnp.float32),
            pltpu.VMEM_SHARED((N, W), jnp.float32),
            pltpu.SemaphoreType.DMA,
            pltpu.SemaphoreType.DMA,
            pltpu.SemaphoreType.DMA,
            pltpu.SemaphoreType.DMA,
            pltpu.SemaphoreType.DMA,
            pltpu.SemaphoreType.DMA,
            pltpu.SemaphoreType.DMA,
            pltpu.SemaphoreType.DMA,
        ],
    )
    def spmm(table, gidx, sidx, ztab, out, gbuf, sbuf, rows, acc,
             gs0, gs1, gs2, gs3, ss0, ss1, ss2, ss3):
        cid = lax.axis_index("c")
        sid = lax.axis_index("s")
        wid = sid * NC + cid
        gs = (gs0, gs1, gs2, gs3)
        ss = (ss0, ss1, ss2, ss3)

        for k in range(KZ):
            blk = sid + k * NS

            @pl.when(blk < NZB)
            def _():
                pltpu.sync_copy(ztab.at[pl.ds(blk * ZR, ZR)],
                                acc.at[pl.ds(blk * ZR, ZR)])

        plsc.subcore_barrier()

        def w_gather(c, b):
            pltpu.make_async_copy(table.at[gbuf.at[c]], rows.at[b],
                                  gs[b]).wait()

        def i_gather(c, b):
            pltpu.async_copy(table.at[gbuf.at[c]], rows.at[b], gs[b])

        def i_scatter(c, b):
            pltpu.async_copy(rows.at[b], acc.at[sbuf.at[c]], ss[b], add=True)

        def w_scatter(c, b):
            pltpu.make_async_copy(rows.at[b], acc.at[sbuf.at[c]],
                                  ss[b]).wait()

        for g in range(NGROUP):
            pltpu.sync_copy(gidx.at[wid, g], gbuf)
            pltpu.sync_copy(sidx.at[wid, g], sbuf)
            for j in range(RING - 1):
                i_gather(j, j)

            def chunk_body(t, carry):
                for k in range(RING):
                    c = t * RING + k
                    w_gather(c, k)
                    i_scatter(c, k)
                    bR = (k + RING - 1) % RING

                    @pl.when(c >= 1)
                    def _():
                        w_scatter(c - 1, bR)

                    @pl.when(c + RING - 1 < G)
                    def _():
                        i_gather(c + RING - 1, bR)
                return carry
            lax.fori_loop(0, G // RING, chunk_body, 0)

            ct = G - 1
            w_gather(ct, 0)
            i_scatter(ct, 0)
            w_scatter(ct - 1, RING - 1)
            w_scatter(ct, 0)

        plsc.subcore_barrier()
        for k in range(KZ):
            blk = sid + k * NS

            @pl.when(blk < NZB)
            def _():
                pltpu.sync_copy(acc.at[pl.ds(blk * ZR, ZR)],
                                out.at[cid, pl.ds(blk * ZR, ZR)])

    return spmm


def _make_sc_pre(V, E, N, B):
    W = 128
    EPW = E // NW
    NCHUNK = EPW // B
    RING = 4
    G = 25
    assert NCHUNK % G == 0 and G % RING == 1
    NGROUP = NCHUNK // G
    ZR = 200
    NZB = N // ZR
    KZ = (NZB + NS - 1) // NS

    mesh = plsc.VectorSubcoreMesh(core_axis_name="c", subcore_axis_name="s", num_cores=NC, num_subcores=NS)

    @functools.partial(
        pl.kernel,
        mesh=mesh,
        out_type=(jax.ShapeDtypeStruct((NC, N, W), jnp.float32),
                  jax.ShapeDtypeStruct((NC, N, W), jnp.float32)),
        scratch_types=[
            pltpu.VMEM((G, B), jnp.int32),
            pltpu.VMEM((G, B), jnp.int32),
            pltpu.VMEM((4, B, W), jnp.float32),
            pltpu.VMEM_SHARED((N, W), jnp.float32),
            pltpu.SemaphoreType.DMA,
            pltpu.SemaphoreType.DMA,
            pltpu.SemaphoreType.DMA,
            pltpu.SemaphoreType.DMA,
            pltpu.SemaphoreType.DMA,
            pltpu.SemaphoreType.DMA,
            pltpu.SemaphoreType.DMA,
            pltpu.SemaphoreType.DMA,
        ],
    )
    def prek(table, gidx, sidx, ztab, out1, out2, gbuf, sbuf, rows, acc,
             gs0, gs1, gs2, gs3, ss0, ss1, ss2, ss3):
        cid = lax.axis_index("c")
        sid = lax.axis_index("s")
        wid = sid * NC + cid
        gs = (gs0, gs1, gs2, gs3)
        ss = (ss0, ss1, ss2, ss3)

        def zero_acc():
            for k in range(KZ):
                blk = sid + k * NS

                @pl.when(blk < NZB)
                def _():
                    pltpu.sync_copy(ztab.at[pl.ds(blk * ZR, ZR)],
                                    acc.at[pl.ds(blk * ZR, ZR)])

        def copy_out(out):
            for k in range(KZ):
                blk = sid + k * NS

                @pl.when(blk < NZB)
                def _():
                    pltpu.sync_copy(acc.at[pl.ds(blk * ZR, ZR)],
                                    out.at[cid, pl.ds(blk * ZR, ZR)])

        def w_gather(c, b):
            pltpu.make_async_copy(table.at[gbuf.at[c]], rows.at[b],
                                  gs[b]).wait()

        def i_gather(c, b):
            pltpu.async_copy(table.at[gbuf.at[c]], rows.at[b], gs[b])

        def i_scatter(c, b):
            pltpu.async_copy(rows.at[b], acc.at[sbuf.at[c]], ss[b], add=True)

        def w_scatter(c, b):
            pltpu.make_async_copy(rows.at[b], acc.at[sbuf.at[c]],
                                  ss[b]).wait()

        zero_acc()
        plsc.subcore_barrier()

        for g in range(NGROUP):
            pltpu.sync_copy(gidx.at[wid, g], gbuf)
            pltpu.sync_copy(sidx.at[wid, g], sbuf)
            for j in range(RING - 1):
                i_gather(j, j)

            def chunk_body(t, carry):
                for k in range(RING):
                    c = t * RING + k
                    w_gather(c, k)
                    i_scatter(c, k)
                    bR = (k + RING - 1) % RING

                    @pl.when(c >= 1)
                    def _():
                        w_scatter(c - 1, bR)

                    @pl.when(c + RING - 1 < G)
                    def _():
                        i_gather(c + RING - 1, bR)
                return carry
            lax.fori_loop(0, G // RING, chunk_body, 0)

            ct = G - 1
            w_gather(ct, 0)
            i_scatter(ct, 0)
            w_scatter(ct - 1, RING - 1)
            w_scatter(ct, 0)

        plsc.subcore_barrier()
        copy_out(out1)
        plsc.subcore_barrier()

        zero_acc()
        one16 = jnp.ones((16,), jnp.float32)
        for i in range(B):
            for j in range(W // 16):
                rows[0, i, pl.ds(16 * j, 16)] = one16
        plsc.subcore_barrier()

        def i_scatter1(c, b):
            pltpu.async_copy(rows.at[0], acc.at[sbuf.at[c]], ss[b], add=True)

        def w_scatter1(c, b):
            pltpu.make_async_copy(rows.at[0], acc.at[sbuf.at[c]],
                                  ss[b]).wait()

        for g in range(NGROUP):
            pltpu.sync_copy(sidx.at[wid, g], sbuf)

            def deg_body(t, carry):
                for k in range(2):
                    c = t * 2 + k

                    @pl.when(c >= 2)
                    def _():
                        w_scatter1(c - 2, k)

                    i_scatter1(c, k)
                return carry
            lax.fori_loop(0, G // 2, deg_body, 0)

            w_scatter1(G - 2, 0)
            w_scatter1(G - 1, 1)

        plsc.subcore_barrier()
        copy_out(out2)

    return prek


def _l2n(x):
    return x / (jnp.sqrt(jnp.sum(x * x, axis=1, keepdims=True)) + _EPS)


def _tc_pre_body(crel_ref, norm_ref, wn_ref, ent_ref, h0_ref, crnw_ref):
    h0_ref[...] = _l2n(ent_ref[...])
    c = (crel_ref[0] + crel_ref[1]) * norm_ref[...]
    crnw_ref[0] = jnp.dot(c, wn_ref[0], preferred_element_type=jnp.float32)
    crnw_ref[1] = jnp.dot(c, wn_ref[1], preferred_element_type=jnp.float32)


def _tc_precompute(crel_p, norm, w_neigh, emb_ent):
    N, H = emb_ent.shape
    RB = 2000
    grid = (N // RB,)
    return pl.pallas_call(
        _tc_pre_body,
        grid=grid,
        in_specs=[
            pl.BlockSpec((2, RB, H), lambda i: (0, i, 0)),
            pl.BlockSpec((RB, 1), lambda i: (i, 0)),
            pl.BlockSpec((2, H, H), lambda i: (0, 0, 0)),
            pl.BlockSpec((RB, H), lambda i: (i, 0)),
        ],
        out_specs=[
            pl.BlockSpec((RB, H), lambda i: (i, 0)),
            pl.BlockSpec((2, RB, H), lambda i: (0, i, 0)),
        ],
        out_shape=[
            jax.ShapeDtypeStruct((N, H), jnp.float32),
            jax.ShapeDtypeStruct((2, N, H), jnp.float32),
        ],
    )(crel_p, norm, w_neigh, emb_ent)


def _tc_layer_body(y_ref, norm_ref, crnw_ref, x_ref, wn_ref, ws_ref, o_ref):
    ysum = y_ref[0] + y_ref[1]
    z = jnp.dot(ysum, wn_ref[...], preferred_element_type=jnp.float32)
    z = z * norm_ref[...] - crnw_ref[...]
    z = z + jnp.dot(x_ref[...], ws_ref[...], preferred_element_type=jnp.float32)
    o_ref[...] = jnp.where(z >= 0, z, _SLOPE * z)


def _tc_layer(y_p, norm, crnw_l, x, wn_l, ws_l):
    N, H = x.shape
    RB = 2000
    grid = (N // RB,)
    return pl.pallas_call(
        _tc_layer_body,
        grid=grid,
        in_specs=[
            pl.BlockSpec((2, RB, H), lambda i: (0, i, 0)),
            pl.BlockSpec((RB, 1), lambda i: (i, 0)),
            pl.BlockSpec((RB, H), lambda i: (i, 0)),
            pl.BlockSpec((RB, H), lambda i: (i, 0)),
            pl.BlockSpec((H, H), lambda i: (0, 0)),
            pl.BlockSpec((H, H), lambda i: (0, 0)),
        ],
        out_specs=pl.BlockSpec((RB, H), lambda i: (i, 0)),
        out_shape=jax.ShapeDtypeStruct((N, H), jnp.float32),
    )(y_p, norm, crnw_l, x, wn_l, ws_l)


def _tc_layer_gru_body(y_ref, norm_ref, crnw_ref, x_ref, wn_ref, ws_ref,
                       h_ref, wih_ref, bih_ref, whh_ref, bhh_ref, o_ref):
    H = x_ref.shape[1]
    ysum = y_ref[0] + y_ref[1]
    z = jnp.dot(ysum, wn_ref[...], preferred_element_type=jnp.float32)
    z = z * norm_ref[...] - crnw_ref[...]
    z = z + jnp.dot(x_ref[...], ws_ref[...], preferred_element_type=jnp.float32)
    z = jnp.where(z >= 0, z, _SLOPE * z)
    xn = _l2n(z)
    h = h_ref[...]
    gi = lax.dot_general(xn, wih_ref[...], (((1,), (1,)), ((), ())),
                         preferred_element_type=jnp.float32) + bih_ref[...]
    gh = lax.dot_general(h, whh_ref[...], (((1,), (1,)), ((), ())),
                         preferred_element_type=jnp.float32) + bhh_ref[...]
    rg = jax.nn.sigmoid(gi[:, :H] + gh[:, :H])
    zg = jax.nn.sigmoid(gi[:, H:2 * H] + gh[:, H:2 * H])
    ng = jnp.tanh(gi[:, 2 * H:] + rg * gh[:, 2 * H:])
    hn = (1.0 - zg) * ng + zg * h
    o_ref[...] = _l2n(hn)


def _tc_layer_gru(y_p, norm, crnw_l, x, wn_l, ws_l, h, w_ih, b_ih, w_hh, b_hh):
    N, H = x.shape
    RB = 2000
    grid = (N // RB,)
    return pl.pallas_call(
        _tc_layer_gru_body,
        grid=grid,
        in_specs=[
            pl.BlockSpec((2, RB, H), lambda i: (0, i, 0)),
            pl.BlockSpec((RB, 1), lambda i: (i, 0)),
            pl.BlockSpec((RB, H), lambda i: (i, 0)),
            pl.BlockSpec((RB, H), lambda i: (i, 0)),
            pl.BlockSpec((H, H), lambda i: (0, 0)),
            pl.BlockSpec((H, H), lambda i: (0, 0)),
            pl.BlockSpec((RB, H), lambda i: (i, 0)),
            pl.BlockSpec((3 * H, H), lambda i: (0, 0)),
            pl.BlockSpec((1, 3 * H), lambda i: (0, 0)),
            pl.BlockSpec((3 * H, H), lambda i: (0, 0)),
            pl.BlockSpec((1, 3 * H), lambda i: (0, 0)),
        ],
        out_specs=pl.BlockSpec((RB, H), lambda i: (i, 0)),
        out_shape=jax.ShapeDtypeStruct((N, H), jnp.float32),
    )(y_p, norm, crnw_l, x, wn_l, ws_l, h, w_ih, b_ih, w_hh, b_hh)


def kernel(edge_index, edge_type, emb_ent, emb_rel, W_neigh, W_self,
           gru_W_ih, gru_b_ih, gru_W_hh, gru_b_hh):
    N, H = emb_ent.shape
    E = edge_type.shape[0]
    L = W_neigh.shape[0]
    R = emb_rel.shape[0] // 2
    T = 3

    src = edge_index[0]
    dst = edge_index[1]

    B_MAIN = 80
    G = 25
    NGRP = E // NW // B_MAIN // G
    src3 = src.reshape(NW, NGRP, G, B_MAIN)
    dst3 = dst.reshape(NW, NGRP, G, B_MAIN)

    ztab = jnp.zeros((N, H), jnp.float32)

    KREP = 16
    r_rep = jnp.tile(emb_rel[:R], (KREP, 1))
    et_rep = (jnp.arange(E, dtype=jnp.int32) % KREP) * R + edge_type
    et3 = et_rep.reshape(NW, NGRP, G, B_MAIN)

    pre_pass = _make_sc_pre(KREP * R, E, N, B_MAIN)
    spmm_main = _make_sc_spmm(N, H, E, N, B_MAIN)

    crel_p, deg_p = pre_pass(r_rep, et3, dst3, ztab)
    deg = deg_p[0, :, 0] + deg_p[1, :, 0]
    norm = (1.0 / jnp.maximum(deg, 1.0))[:, None]

    b_ih2 = gru_b_ih.reshape(1, 3 * H)
    b_hh2 = gru_b_hh.reshape(1, 3 * H)

    h, crnw = _tc_precompute(crel_p, norm, W_neigh, emb_ent)

    for _t in range(T):
        x = h
        for l in range(L - 1):
            y_p = spmm_main(x, src3, dst3, ztab)
            x = _tc_layer(y_p, norm, crnw[l], x, W_neigh[l], W_self[l])
        y_p = spmm_main(x, src3, dst3, ztab)
        h = _tc_layer_gru(y_p, norm, crnw[L - 1], x, W_neigh[L - 1],
                          W_self[L - 1], h, gru_W_ih, b_ih2, gru_W_hh, b_hh2)
    return h


# speedup vs baseline: 1.2378x; 1.0466x over previous
import functools

import jax
import jax.numpy as jnp
from jax import lax
from jax.experimental import pallas as pl
from jax.experimental.pallas import tpu as pltpu
from jax.experimental.pallas import tpu_sc as plsc

NC = 2
NS = 16
NW = NC * NS

_SLOPE = (1.0 / 8.0 + 1.0 / 3.0) / 2.0
_EPS = 1e-12


def _make_sc_spmm(V, W, E, N, B):
    EPW = E // NW
    assert EPW * NW == E and EPW % B == 0
    NCHUNK = EPW // B
    RING = 3
    G = 25
    assert NCHUNK % G == 0 and G % RING == 1
    NGROUP = NCHUNK // G
    ZR = 200
    NZB = N // ZR
    assert NZB * ZR == N
    KZ = (NZB + NS - 1) // NS

    mesh = plsc.VectorSubcoreMesh(core_axis_name="c", subcore_axis_name="s", num_cores=NC, num_subcores=NS)

    @functools.partial(
        pl.kernel,
        mesh=mesh,
        out_type=jax.ShapeDtypeStruct((NC, N, W), jnp.float32),
        scratch_types=[
            pltpu.VMEM((G, B), jnp.int32),
            pltpu.VMEM((G, B), jnp.int32),
            pltpu.VMEM((3, B, W), jnp.float32),
            pltpu.VMEM_SHARED((N, W), jnp.float32),
            pltpu.SemaphoreType.DMA,
            pltpu.SemaphoreType.DMA,
            pltpu.SemaphoreType.DMA,
            pltpu.SemaphoreType.DMA,
            pltpu.SemaphoreType.DMA,
            pltpu.SemaphoreType.DMA,
            pltpu.SemaphoreType.DMA,
            pltpu.SemaphoreType.DMA,
        ],
    )
    def spmm(table, gidx, sidx, ztab, out, gbuf, sbuf, rows, acc,
             gs0, gs1, gs2, gs3, ss0, ss1, ss2, ss3):
        cid = lax.axis_index("c")
        sid = lax.axis_index("s")
        wid = sid * NC + cid
        gs = (gs0, gs1, gs2, gs3)
        ss = (ss0, ss1, ss2, ss3)

        for k in range(KZ):
            blk = sid + k * NS

            @pl.when(blk < NZB)
            def _():
                pltpu.sync_copy(ztab.at[pl.ds(blk * ZR, ZR)],
                                acc.at[pl.ds(blk * ZR, ZR)])

        plsc.subcore_barrier()

        def w_gather(c, b):
            pltpu.make_async_copy(table.at[gbuf.at[c]], rows.at[b],
                                  gs[b]).wait()

        def i_gather(c, b):
            pltpu.async_copy(table.at[gbuf.at[c]], rows.at[b], gs[b])

        def i_scatter(c, b):
            pltpu.async_copy(rows.at[b], acc.at[sbuf.at[c]], ss[b], add=True)

        def w_scatter(c, b):
            pltpu.make_async_copy(rows.at[b], acc.at[sbuf.at[c]],
                                  ss[b]).wait()

        for g in range(NGROUP):
            pltpu.sync_copy(gidx.at[wid, g], gbuf)
            pltpu.sync_copy(sidx.at[wid, g], sbuf)
            for j in range(RING - 1):
                i_gather(j, j)

            def chunk_body(t, carry):
                for k in range(RING):
                    c = t * RING + k
                    w_gather(c, k)
                    i_scatter(c, k)
                    bR = (k + RING - 1) % RING

                    @pl.when(c >= 1)
                    def _():
                        w_scatter(c - 1, bR)

                    @pl.when(c + RING - 1 < G)
                    def _():
                        i_gather(c + RING - 1, bR)
                return carry
            lax.fori_loop(0, G // RING, chunk_body, 0)

            ct = G - 1
            w_gather(ct, 0)
            i_scatter(ct, 0)
            w_scatter(ct - 1, RING - 1)
            w_scatter(ct, 0)

        plsc.subcore_barrier()
        for k in range(KZ):
            blk = sid + k * NS

            @pl.when(blk < NZB)
            def _():
                pltpu.sync_copy(acc.at[pl.ds(blk * ZR, ZR)],
                                out.at[cid, pl.ds(blk * ZR, ZR)])

    return spmm


def _make_sc_pre(V, E, N, B):
    W = 128
    EPW = E // NW
    NCHUNK = EPW // B
    RING = 3
    G = 25
    assert NCHUNK % G == 0 and G % RING == 1
    NGROUP = NCHUNK // G
    ZR = 200
    NZB = N // ZR
    KZ = (NZB + NS - 1) // NS

    mesh = plsc.VectorSubcoreMesh(core_axis_name="c", subcore_axis_name="s", num_cores=NC, num_subcores=NS)

    @functools.partial(
        pl.kernel,
        mesh=mesh,
        out_type=(jax.ShapeDtypeStruct((NC, N, W), jnp.float32),
                  jax.ShapeDtypeStruct((NC, N, W), jnp.float32)),
        scratch_types=[
            pltpu.VMEM((G, B), jnp.int32),
            pltpu.VMEM((G, B), jnp.int32),
            pltpu.VMEM((3, B, W), jnp.float32),
            pltpu.VMEM_SHARED((N, W), jnp.float32),
            pltpu.SemaphoreType.DMA,
            pltpu.SemaphoreType.DMA,
            pltpu.SemaphoreType.DMA,
            pltpu.SemaphoreType.DMA,
            pltpu.SemaphoreType.DMA,
            pltpu.SemaphoreType.DMA,
            pltpu.SemaphoreType.DMA,
            pltpu.SemaphoreType.DMA,
        ],
    )
    def prek(table, gidx, sidx, ztab, out1, out2, gbuf, sbuf, rows, acc,
             gs0, gs1, gs2, gs3, ss0, ss1, ss2, ss3):
        cid = lax.axis_index("c")
        sid = lax.axis_index("s")
        wid = sid * NC + cid
        gs = (gs0, gs1, gs2, gs3)
        ss = (ss0, ss1, ss2, ss3)

        def zero_acc():
            for k in range(KZ):
                blk = sid + k * NS

                @pl.when(blk < NZB)
                def _():
                    pltpu.sync_copy(ztab.at[pl.ds(blk * ZR, ZR)],
                                    acc.at[pl.ds(blk * ZR, ZR)])

        def copy_out(out):
            for k in range(KZ):
                blk = sid + k * NS

                @pl.when(blk < NZB)
                def _():
                    pltpu.sync_copy(acc.at[pl.ds(blk * ZR, ZR)],
                                    out.at[cid, pl.ds(blk * ZR, ZR)])

        def w_gather(c, b):
            pltpu.make_async_copy(table.at[gbuf.at[c]], rows.at[b],
                                  gs[b]).wait()

        def i_gather(c, b):
            pltpu.async_copy(table.at[gbuf.at[c]], rows.at[b], gs[b])

        def i_scatter(c, b):
            pltpu.async_copy(rows.at[b], acc.at[sbuf.at[c]], ss[b], add=True)

        def w_scatter(c, b):
            pltpu.make_async_copy(rows.at[b], acc.at[sbuf.at[c]],
                                  ss[b]).wait()

        zero_acc()
        plsc.subcore_barrier()

        for g in range(NGROUP):
            pltpu.sync_copy(gidx.at[wid, g], gbuf)
            pltpu.sync_copy(sidx.at[wid, g], sbuf)
            for j in range(RING - 1):
                i_gather(j, j)

            def chunk_body(t, carry):
                for k in range(RING):
                    c = t * RING + k
                    w_gather(c, k)
                    i_scatter(c, k)
                    bR = (k + RING - 1) % RING

                    @pl.when(c >= 1)
                    def _():
                        w_scatter(c - 1, bR)

                    @pl.when(c + RING - 1 < G)
                    def _():
                        i_gather(c + RING - 1, bR)
                return carry
            lax.fori_loop(0, G // RING, chunk_body, 0)

            ct = G - 1
            w_gather(ct, 0)
            i_scatter(ct, 0)
            w_scatter(ct - 1, RING - 1)
            w_scatter(ct, 0)

        plsc.subcore_barrier()
        copy_out(out1)
        plsc.subcore_barrier()

        zero_acc()
        one16 = jnp.ones((16,), jnp.float32)
        for i in range(B):
            for j in range(W // 16):
                rows[0, i, pl.ds(16 * j, 16)] = one16
        plsc.subcore_barrier()

        def i_scatter1(c, b):
            pltpu.async_copy(rows.at[0], acc.at[sbuf.at[c]], ss[b], add=True)

        def w_scatter1(c, b):
            pltpu.make_async_copy(rows.at[0], acc.at[sbuf.at[c]],
                                  ss[b]).wait()

        for g in range(NGROUP):
            pltpu.sync_copy(sidx.at[wid, g], sbuf)

            def deg_body(t, carry):
                for k in range(2):
                    c = t * 2 + k

                    @pl.when(c >= 2)
                    def _():
                        w_scatter1(c - 2, k)

                    i_scatter1(c, k)
                return carry
            lax.fori_loop(0, G // 2, deg_body, 0)

            w_scatter1(G - 2, 0)
            w_scatter1(G - 1, 1)

        plsc.subcore_barrier()
        copy_out(out2)

    return prek


def _l2n(x):
    return x / (jnp.sqrt(jnp.sum(x * x, axis=1, keepdims=True)) + _EPS)


def _tc_pre_body(crel_ref, norm_ref, wn_ref, ent_ref, h0_ref, crnw_ref):
    h0_ref[...] = _l2n(ent_ref[...])
    c = (crel_ref[0] + crel_ref[1]) * norm_ref[...]
    crnw_ref[0] = jnp.dot(c, wn_ref[0], preferred_element_type=jnp.float32)
    crnw_ref[1] = jnp.dot(c, wn_ref[1], preferred_element_type=jnp.float32)


def _tc_precompute(crel_p, norm, w_neigh, emb_ent):
    N, H = emb_ent.shape
    RB = 2000
    grid = (N // RB,)
    return pl.pallas_call(
        _tc_pre_body,
        grid=grid,
        in_specs=[
            pl.BlockSpec((2, RB, H), lambda i: (0, i, 0)),
            pl.BlockSpec((RB, 1), lambda i: (i, 0)),
            pl.BlockSpec((2, H, H), lambda i: (0, 0, 0)),
            pl.BlockSpec((RB, H), lambda i: (i, 0)),
        ],
        out_specs=[
            pl.BlockSpec((RB, H), lambda i: (i, 0)),
            pl.BlockSpec((2, RB, H), lambda i: (0, i, 0)),
        ],
        out_shape=[
            jax.ShapeDtypeStruct((N, H), jnp.float32),
            jax.ShapeDtypeStruct((2, N, H), jnp.float32),
        ],
    )(crel_p, norm, w_neigh, emb_ent)


def _tc_layer_body(y_ref, norm_ref, crnw_ref, x_ref, wn_ref, ws_ref, o_ref):
    ysum = y_ref[0] + y_ref[1]
    z = jnp.dot(ysum, wn_ref[...], preferred_element_type=jnp.float32)
    z = z * norm_ref[...] - crnw_ref[...]
    z = z + jnp.dot(x_ref[...], ws_ref[...], preferred_element_type=jnp.float32)
    o_ref[...] = jnp.where(z >= 0, z, _SLOPE * z)


def _tc_layer(y_p, norm, crnw_l, x, wn_l, ws_l):
    N, H = x.shape
    RB = 2000
    grid = (N // RB,)
    return pl.pallas_call(
        _tc_layer_body,
        grid=grid,
        in_specs=[
            pl.BlockSpec((2, RB, H), lambda i: (0, i, 0)),
            pl.BlockSpec((RB, 1), lambda i: (i, 0)),
            pl.BlockSpec((RB, H), lambda i: (i, 0)),
            pl.BlockSpec((RB, H), lambda i: (i, 0)),
            pl.BlockSpec((H, H), lambda i: (0, 0)),
            pl.BlockSpec((H, H), lambda i: (0, 0)),
        ],
        out_specs=pl.BlockSpec((RB, H), lambda i: (i, 0)),
        out_shape=jax.ShapeDtypeStruct((N, H), jnp.float32),
    )(y_p, norm, crnw_l, x, wn_l, ws_l)


def _tc_layer_gru_body(y_ref, norm_ref, crnw_ref, x_ref, wn_ref, ws_ref,
                       h_ref, wih_ref, bih_ref, whh_ref, bhh_ref, o_ref):
    H = x_ref.shape[1]
    ysum = y_ref[0] + y_ref[1]
    z = jnp.dot(ysum, wn_ref[...], preferred_element_type=jnp.float32)
    z = z * norm_ref[...] - crnw_ref[...]
    z = z + jnp.dot(x_ref[...], ws_ref[...], preferred_element_type=jnp.float32)
    z = jnp.where(z >= 0, z, _SLOPE * z)
    xn = _l2n(z)
    h = h_ref[...]
    gi = lax.dot_general(xn, wih_ref[...], (((1,), (1,)), ((), ())),
                         preferred_element_type=jnp.float32) + bih_ref[...]
    gh = lax.dot_general(h, whh_ref[...], (((1,), (1,)), ((), ())),
                         preferred_element_type=jnp.float32) + bhh_ref[...]
    rg = jax.nn.sigmoid(gi[:, :H] + gh[:, :H])
    zg = jax.nn.sigmoid(gi[:, H:2 * H] + gh[:, H:2 * H])
    ng = jnp.tanh(gi[:, 2 * H:] + rg * gh[:, 2 * H:])
    hn = (1.0 - zg) * ng + zg * h
    o_ref[...] = _l2n(hn)


def _tc_layer_gru(y_p, norm, crnw_l, x, wn_l, ws_l, h, w_ih, b_ih, w_hh, b_hh):
    N, H = x.shape
    RB = 2000
    grid = (N // RB,)
    return pl.pallas_call(
        _tc_layer_gru_body,
        grid=grid,
        in_specs=[
            pl.BlockSpec((2, RB, H), lambda i: (0, i, 0)),
            pl.BlockSpec((RB, 1), lambda i: (i, 0)),
            pl.BlockSpec((RB, H), lambda i: (i, 0)),
            pl.BlockSpec((RB, H), lambda i: (i, 0)),
            pl.BlockSpec((H, H), lambda i: (0, 0)),
            pl.BlockSpec((H, H), lambda i: (0, 0)),
            pl.BlockSpec((RB, H), lambda i: (i, 0)),
            pl.BlockSpec((3 * H, H), lambda i: (0, 0)),
            pl.BlockSpec((1, 3 * H), lambda i: (0, 0)),
            pl.BlockSpec((3 * H, H), lambda i: (0, 0)),
            pl.BlockSpec((1, 3 * H), lambda i: (0, 0)),
        ],
        out_specs=pl.BlockSpec((RB, H), lambda i: (i, 0)),
        out_shape=jax.ShapeDtypeStruct((N, H), jnp.float32),
    )(y_p, norm, crnw_l, x, wn_l, ws_l, h, w_ih, b_ih, w_hh, b_hh)


def kernel(edge_index, edge_type, emb_ent, emb_rel, W_neigh, W_self,
           gru_W_ih, gru_b_ih, gru_W_hh, gru_b_hh):
    N, H = emb_ent.shape
    E = edge_type.shape[0]
    L = W_neigh.shape[0]
    R = emb_rel.shape[0] // 2
    T = 3

    src = edge_index[0]
    dst = edge_index[1]

    B_MAIN = 100
    G = 25
    NGRP = E // NW // B_MAIN // G
    src3 = src.reshape(NW, NGRP, G, B_MAIN)
    dst3 = dst.reshape(NW, NGRP, G, B_MAIN)

    ztab = jnp.zeros((N, H), jnp.float32)

    KREP = 16
    r_rep = jnp.tile(emb_rel[:R], (KREP, 1))
    et_rep = (jnp.arange(E, dtype=jnp.int32) % KREP) * R + edge_type
    et3 = et_rep.reshape(NW, NGRP, G, B_MAIN)

    pre_pass = _make_sc_pre(KREP * R, E, N, B_MAIN)
    spmm_main = _make_sc_spmm(N, H, E, N, B_MAIN)

    crel_p, deg_p = pre_pass(r_rep, et3, dst3, ztab)
    deg = deg_p[0, :, 0] + deg_p[1, :, 0]
    norm = (1.0 / jnp.maximum(deg, 1.0))[:, None]

    b_ih2 = gru_b_ih.reshape(1, 3 * H)
    b_hh2 = gru_b_hh.reshape(1, 3 * H)

    h, crnw = _tc_precompute(crel_p, norm, W_neigh, emb_ent)

    for _t in range(T):
        x = h
        for l in range(L - 1):
            y_p = spmm_main(x, src3, dst3, ztab)
            x = _tc_layer(y_p, norm, crnw[l], x, W_neigh[l], W_self[l])
        y_p = spmm_main(x, src3, dst3, ztab)
        h = _tc_layer_gru(y_p, norm, crnw[L - 1], x, W_neigh[L - 1],
                          W_self[L - 1], h, gru_W_ih, b_ih2, gru_W_hh, b_hh2)
    return h
